# baseline (speedup 1.0000x reference)
import jax
import jax.numpy as jnp
from jax.experimental import pallas as pl
from jax.experimental.pallas import tpu as pltpu

_C = 1024
_NBUF = 5
_D = 768


def _body(in_hbm, pos_hbm, out_hbm, ibuf, obuf, posv, isems, osems, psems):
    n_rows = in_hbm.shape[0]
    seq = pos_hbm.shape[0]
    n_chunks = n_rows // _C
    pos_chunks = seq // _C

    def in_copy(c):
        return pltpu.make_async_copy(
            in_hbm.at[pl.ds(c * _C, _C)], ibuf.at[c % _NBUF], isems.at[c % _NBUF]
        )

    def out_copy(c):
        return pltpu.make_async_copy(
            obuf.at[c % _NBUF], out_hbm.at[pl.ds(c * _C, _C)], osems.at[c % _NBUF]
        )

    def pos_copy(p):
        return pltpu.make_async_copy(
            pos_hbm.at[pl.ds(p * _C, _C)], posv.at[pl.ds(p * _C, _C)], psems.at[p]
        )

    pos_copy(0).start()
    for k in range(_NBUF):
        in_copy(k).start()
        if k + 1 < pos_chunks:
            pos_copy(k + 1).start()
    for p in range(_NBUF + 1, pos_chunks):
        pos_copy(p).start()

    for c in range(n_chunks):
        slot = c % _NBUF
        if c >= _NBUF:
            out_copy(c - _NBUF).wait()
        in_copy(c).wait()
        if c < pos_chunks:
            pos_copy(c).wait()
        obuf[slot] = ibuf[slot] + posv[pl.ds((c * _C) % seq, _C)]
        out_copy(c).start()
        if c + _NBUF < n_chunks:
            in_copy(c + _NBUF).start()

    for c in range(n_chunks - _NBUF, n_chunks):
        out_copy(c).wait()


def kernel(input_embeds, pos_table):
    batch, seq, d = input_embeds.shape
    flat = input_embeds.reshape(batch * seq, d)

    out = pl.pallas_call(
        _body,
        in_specs=[
            pl.BlockSpec(memory_space=pl.ANY),
            pl.BlockSpec(memory_space=pl.ANY),
        ],
        out_specs=pl.BlockSpec(memory_space=pl.ANY),
        out_shape=jax.ShapeDtypeStruct((batch * seq, d), input_embeds.dtype),
        scratch_shapes=[
            pltpu.VMEM((_NBUF, _C, _D), jnp.float32),
            pltpu.VMEM((_NBUF, _C, _D), jnp.float32),
            pltpu.VMEM((8192, _D), jnp.float32),
            pltpu.SemaphoreType.DMA((_NBUF,)),
            pltpu.SemaphoreType.DMA((_NBUF,)),
            pltpu.SemaphoreType.DMA((8192 // _C,)),
        ],
    )(flat, pos_table)
    return out.reshape(batch, seq, d)


# speedup vs baseline: 1.0020x; 1.0020x over previous
import jax
import jax.numpy as jnp
from jax.experimental import pallas as pl
from jax.experimental.pallas import tpu as pltpu

_C = 512
_NBUF = 8
_D = 768


def _body(in_hbm, pos_hbm, out_hbm, ibuf, obuf, posv, isems, osems, psems):
    n_rows = in_hbm.shape[0]
    seq = pos_hbm.shape[0]
    n_chunks = n_rows // _C
    pos_chunks = seq // _C

    def in_copy(c):
        return pltpu.make_async_copy(
            in_hbm.at[pl.ds(c * _C, _C)], ibuf.at[c % _NBUF], isems.at[c % _NBUF]
        )

    def out_copy(c):
        return pltpu.make_async_copy(
            obuf.at[c % _NBUF], out_hbm.at[pl.ds(c * _C, _C)], osems.at[c % _NBUF]
        )

    def pos_copy(p):
        return pltpu.make_async_copy(
            pos_hbm.at[pl.ds(p * _C, _C)], posv.at[pl.ds(p * _C, _C)], psems.at[p]
        )

    pos_copy(0).start()
    for k in range(_NBUF):
        in_copy(k).start()
        if k + 1 < pos_chunks:
            pos_copy(k + 1).start()
    for p in range(_NBUF + 1, pos_chunks):
        pos_copy(p).start()

    for c in range(n_chunks):
        slot = c % _NBUF
        if c >= _NBUF:
            out_copy(c - _NBUF).wait()
        in_copy(c).wait()
        if c < pos_chunks:
            pos_copy(c).wait()
        obuf[slot] = ibuf[slot] + posv[pl.ds((c * _C) % seq, _C)]
        out_copy(c).start()
        if c + _NBUF < n_chunks:
            in_copy(c + _NBUF).start()

    for c in range(n_chunks - _NBUF, n_chunks):
        out_copy(c).wait()


def kernel(input_embeds, pos_table):
    batch, seq, d = input_embeds.shape
    flat = input_embeds.reshape(batch * seq, d)

    out = pl.pallas_call(
        _body,
        in_specs=[
            pl.BlockSpec(memory_space=pl.ANY),
            pl.BlockSpec(memory_space=pl.ANY),
        ],
        out_specs=pl.BlockSpec(memory_space=pl.ANY),
        out_shape=jax.ShapeDtypeStruct((batch * seq, d), input_embeds.dtype),
        scratch_shapes=[
            pltpu.VMEM((_NBUF, _C, _D), jnp.float32),
            pltpu.VMEM((_NBUF, _C, _D), jnp.float32),
            pltpu.VMEM((8192, _D), jnp.float32),
            pltpu.SemaphoreType.DMA((_NBUF,)),
            pltpu.SemaphoreType.DMA((_NBUF,)),
            pltpu.SemaphoreType.DMA((8192 // _C,)),
        ],
    )(flat, pos_table)
    return out.reshape(batch, seq, d)


# speedup vs baseline: 1.0075x; 1.0055x over previous
import jax
import jax.numpy as jnp
from jax.experimental import pallas as pl
from jax.experimental.pallas import tpu as pltpu

_C = 1024
_NBUF = 4
_D = 768


def _body(in_hbm, pos_hbm, out_hbm, ibuf, obuf, posv, isems, osems, psems):
    n_rows = in_hbm.shape[0]
    seq = pos_hbm.shape[0]
    n_chunks = n_rows // _C
    pos_chunks = seq // _C

    def in_copy(c):
        return pltpu.make_async_copy(
            in_hbm.at[pl.ds(c * _C, _C)], ibuf.at[c % _NBUF], isems.at[c % _NBUF]
        )

    def out_copy(c):
        return pltpu.make_async_copy(
            obuf.at[c % _NBUF], out_hbm.at[pl.ds(c * _C, _C)], osems.at[c % _NBUF]
        )

    def pos_copy(p):
        return pltpu.make_async_copy(
            pos_hbm.at[pl.ds(p * _C, _C)], posv.at[pl.ds(p * _C, _C)], psems.at[p]
        )

    pos_copy(0).start()
    for k in range(_NBUF):
        in_copy(k).start()
        if k + 1 < pos_chunks:
            pos_copy(k + 1).start()
    for p in range(_NBUF + 1, pos_chunks):
        pos_copy(p).start()

    for c in range(n_chunks):
        slot = c % _NBUF
        if c >= _NBUF:
            out_copy(c - _NBUF).wait()
        in_copy(c).wait()
        if c < pos_chunks:
            pos_copy(c).wait()
        obuf[slot] = ibuf[slot] + posv[pl.ds((c * _C) % seq, _C)]
        out_copy(c).start()
        if c + _NBUF < n_chunks:
            in_copy(c + _NBUF).start()

    for c in range(n_chunks - _NBUF, n_chunks):
        out_copy(c).wait()


def kernel(input_embeds, pos_table):
    batch, seq, d = input_embeds.shape
    flat = input_embeds.reshape(batch * seq, d)

    out = pl.pallas_call(
        _body,
        in_specs=[
            pl.BlockSpec(memory_space=pl.ANY),
            pl.BlockSpec(memory_space=pl.ANY),
        ],
        out_specs=pl.BlockSpec(memory_space=pl.ANY),
        out_shape=jax.ShapeDtypeStruct((batch * seq, d), input_embeds.dtype),
        scratch_shapes=[
            pltpu.VMEM((_NBUF, _C, _D), jnp.float32),
            pltpu.VMEM((_NBUF, _C, _D), jnp.float32),
            pltpu.VMEM((8192, _D), jnp.float32),
            pltpu.SemaphoreType.DMA((_NBUF,)),
            pltpu.SemaphoreType.DMA((_NBUF,)),
            pltpu.SemaphoreType.DMA((8192 // _C,)),
        ],
    )(flat, pos_table)
    return out.reshape(batch, seq, d)
